# baseline (speedup 1.0000x reference)
import jax
import jax.numpy as jnp
from jax import lax
from jax.experimental import pallas as pl
from jax.experimental.pallas import tpu as pltpu
from jax.experimental.pallas import tpu_sc as plsc

_NUM_VOCAB = 100000
_EMBED_DIM = 64
_BATCH = 16384

_info = plsc.get_sparse_core_info()
_NC, _NS = _info.num_cores, _info.num_subcores
_NW = _NC * _NS
_BPW = _BATCH // _NW


def _sc_body(word_hbm, context_hbm, encode_hbm, decode_hbm,
             out_w_hbm, out_c_hbm,
             idx_w_v, idx_c_v, rows_w_v, rows_c_v, sem_w, sem_c):
    wid = lax.axis_index("s") * _NC + lax.axis_index("c")
    base = wid * _BPW
    pltpu.sync_copy(word_hbm.at[pl.ds(base, _BPW)], idx_w_v)
    pltpu.sync_copy(context_hbm.at[pl.ds(base, _BPW)], idx_c_v)
    cw = pltpu.async_copy(encode_hbm.at[idx_w_v], rows_w_v, sem_w)
    cc = pltpu.async_copy(decode_hbm.at[idx_c_v], rows_c_v, sem_c)
    cw.wait()
    pltpu.sync_copy(rows_w_v, out_w_hbm.at[pl.ds(base, _BPW)])
    cc.wait()
    pltpu.sync_copy(rows_c_v, out_c_hbm.at[pl.ds(base, _BPW)])


@jax.jit
def _skipgram(word, context, encode_W, decode_W):
    mesh = plsc.VectorSubcoreMesh(core_axis_name="c", subcore_axis_name="s")
    f = pl.kernel(
        _sc_body,
        mesh=mesh,
        out_type=(
            jax.ShapeDtypeStruct((_BATCH, _EMBED_DIM), jnp.float32),
            jax.ShapeDtypeStruct((_BATCH, _EMBED_DIM), jnp.float32),
        ),
        scratch_types=[
            pltpu.VMEM((_BPW,), jnp.int32),
            pltpu.VMEM((_BPW,), jnp.int32),
            pltpu.VMEM((_BPW, _EMBED_DIM), jnp.float32),
            pltpu.VMEM((_BPW, _EMBED_DIM), jnp.float32),
            pltpu.SemaphoreType.DMA,
            pltpu.SemaphoreType.DMA,
        ],
        compiler_params=pltpu.CompilerParams(use_tc_tiling_on_sc=False),
    )
    return f(word, context, encode_W, decode_W)


def kernel(word, context, encode_W, decode_W):
    return _skipgram(word, context, encode_W, decode_W)
